# Initial kernel scaffold; baseline (speedup 1.0000x reference)
import jax
import jax.numpy as jnp
from jax.experimental import pallas as pl


def kernel(pos):
    raise NotImplementedError("write your pallas kernel here")


# speedup vs baseline: 10.2589x; 10.2589x over previous
import functools

import jax
import jax.numpy as jnp
from jax import lax
from jax.experimental import pallas as pl
from jax.experimental.pallas import tpu as pltpu
from jax.experimental.pallas import tpu_sc as plsc

_B, _N, _G, _K = 4, 8192, 512, 32
_NW = 32
_GPW = (_B * _G) // _NW


def _fps_body(x_ref, y_ref, z_ref, cx_ref, cy_ref, cz_ref):
    B, N, G = _B, _N, _G
    x = x_ref[...]
    y = y_ref[...]
    z = z_ref[...]
    iota = lax.broadcasted_iota(jnp.int32, (B, N), 1)
    giota = lax.broadcasted_iota(jnp.int32, (B, G), 1)

    lx = x[:, 0:1]
    ly = y[:, 0:1]
    lz = z[:, 0:1]
    zero = jnp.zeros((B, G), jnp.float32)
    cxa = jnp.where(giota == 0, lx, zero)
    cya = jnp.where(giota == 0, ly, zero)
    cza = jnp.where(giota == 0, lz, zero)
    dists = jnp.full((B, N), jnp.inf, jnp.float32)

    def body(i, carry):
        dists, lx, ly, lz, cxa, cya, cza = carry
        dx = x - lx
        dy = y - ly
        dz = z - lz
        d = (dx * dx + dy * dy) + dz * dz
        dists = jnp.minimum(dists, d)
        m = jnp.max(dists, axis=1, keepdims=True)
        sel = dists == m
        nxt = jnp.min(jnp.where(sel, iota, N), axis=1, keepdims=True)
        onehot = iota == nxt
        lx = jnp.sum(jnp.where(onehot, x, 0.0), axis=1, keepdims=True)
        ly = jnp.sum(jnp.where(onehot, y, 0.0), axis=1, keepdims=True)
        lz = jnp.sum(jnp.where(onehot, z, 0.0), axis=1, keepdims=True)
        gsel = giota == i
        cxa = jnp.where(gsel, lx, cxa)
        cya = jnp.where(gsel, ly, cya)
        cza = jnp.where(gsel, lz, cza)
        return (dists, lx, ly, lz, cxa, cya, cza)

    carry = (dists, lx, ly, lz, cxa, cya, cza)
    carry = lax.fori_loop(1, G, body, carry)
    _, _, _, _, cxa, cya, cza = carry
    cx_ref[...] = cxa
    cy_ref[...] = cya
    cz_ref[...] = cza


def _fps(x, y, z):
    out = jax.ShapeDtypeStruct((_B, _G), jnp.float32)
    return pl.pallas_call(
        _fps_body,
        out_shape=[out, out, out],
    )(x, y, z)


_GT = 128


def _knn_body(cx_ref, cy_ref, cz_ref, px_ref, py_ref, pz_ref, idx_ref):
    N, K, GT, B = _N, _K, _GT, _B
    cx = cx_ref[0]
    cy = cy_ref[0]
    cz = cz_ref[0]
    px = px_ref[0]
    py = py_ref[0]
    pz = pz_ref[0]
    c2 = (cx * cx + cy * cy) + cz * cz
    p2 = (px * px + py * py) + pz * pz
    r = lambda a: a.astype(jnp.bfloat16).astype(jnp.float32)
    cp = (r(cx) * r(px) + r(cy) * r(py)) + r(cz) * r(pz)
    d2 = (c2 + p2) - 2.0 * cp
    iota = lax.broadcasted_iota(jnp.int32, (GT, N), 1)
    inf = jnp.float32(jnp.inf)
    for k in range(K):
        m = jnp.min(d2, axis=1, keepdims=True)
        ik = jnp.min(jnp.where(d2 == m, iota, N), axis=1, keepdims=True)
        idx_ref[0, :, k:k + 1] = ik
        d2 = jnp.where(iota == ik, inf, d2)


def _knn(cxp, cyp, czp, px, py, pz):
    B, N, G, K, GT = _B, _N, _G, _K, _GT
    grid = (B, G // GT)
    cspec = pl.BlockSpec((1, GT, 1), lambda b, t: (b * (G // GT) + t, 0, 0))
    pspec = pl.BlockSpec((1, 1, N), lambda b, t: (b, 0, 0))
    c3 = lambda a: a.reshape(B * G // GT, GT, 1)
    p3 = lambda a: a.reshape(B, 1, N)
    return pl.pallas_call(
        _knn_body,
        grid=grid,
        in_specs=[cspec, cspec, cspec, pspec, pspec, pspec],
        out_specs=pl.BlockSpec((1, GT, K), lambda b, t: (b, t, 0)),
        out_shape=jax.ShapeDtypeStruct((B, G, K), jnp.int32),
    )(c3(cxp), c3(cyp), c3(czp), p3(px), p3(py), p3(pz))


def _sc_group_body(x_h, y_h, z_h, cx_h, cy_h, cz_h, idx_h,
                   ox_h, oy_h, oz_h,
                   xv, yv, zv, idxv, cxv, cyv, czv, outx, outy, outz):
    N, K, GPW = _N, _K, _GPW
    cid = lax.axis_index("c")
    sid = lax.axis_index("s")
    w = sid * 2 + cid
    b = (w * GPW) // _G
    pltpu.sync_copy(x_h.at[pl.ds(b * N, N)], xv)
    pltpu.sync_copy(y_h.at[pl.ds(b * N, N)], yv)
    pltpu.sync_copy(z_h.at[pl.ds(b * N, N)], zv)
    pltpu.sync_copy(idx_h.at[pl.ds(w * GPW * K, GPW * K)], idxv)
    pltpu.sync_copy(cx_h.at[pl.ds(w * GPW * K, GPW * K)], cxv)
    pltpu.sync_copy(cy_h.at[pl.ds(w * GPW * K, GPW * K)], cyv)
    pltpu.sync_copy(cz_h.at[pl.ds(w * GPW * K, GPW * K)], czv)

    def body(v, _):
        off = v * 16
        iv = idxv[pl.ds(off, 16)]
        outx[pl.ds(off, 16)] = plsc.load_gather(xv, [iv]) - cxv[pl.ds(off, 16)]
        outy[pl.ds(off, 16)] = plsc.load_gather(yv, [iv]) - cyv[pl.ds(off, 16)]
        outz[pl.ds(off, 16)] = plsc.load_gather(zv, [iv]) - czv[pl.ds(off, 16)]
        return 0

    lax.fori_loop(0, (GPW * K) // 16, body, 0)
    pltpu.sync_copy(outx, ox_h.at[pl.ds(w * GPW * K, GPW * K)])
    pltpu.sync_copy(outy, oy_h.at[pl.ds(w * GPW * K, GPW * K)])
    pltpu.sync_copy(outz, oz_h.at[pl.ds(w * GPW * K, GPW * K)])


def _sc_group(x, y, z, cx, cy, cz, idx):
    B, N, G, K, GPW = _B, _N, _G, _K, _GPW
    out = jax.ShapeDtypeStruct((B * G * K,), jnp.float32)
    mesh = plsc.VectorSubcoreMesh(core_axis_name="c", subcore_axis_name="s")
    f = pl.kernel(
        _sc_group_body,
        out_type=[out, out, out],
        mesh=mesh,
        compiler_params=pltpu.CompilerParams(needs_layout_passes=False),
        scratch_types=[
            pltpu.VMEM((N,), jnp.float32),
            pltpu.VMEM((N,), jnp.float32),
            pltpu.VMEM((N,), jnp.float32),
            pltpu.VMEM((GPW * K,), jnp.int32),
            pltpu.VMEM((GPW * K,), jnp.float32),
            pltpu.VMEM((GPW * K,), jnp.float32),
            pltpu.VMEM((GPW * K,), jnp.float32),
            pltpu.VMEM((GPW * K,), jnp.float32),
            pltpu.VMEM((GPW * K,), jnp.float32),
            pltpu.VMEM((GPW * K,), jnp.float32),
        ],
    )
    exp = lambda a: jnp.broadcast_to(
        a.reshape(B * G, 1), (B * G, K)).reshape(-1)
    return f(x.reshape(-1), y.reshape(-1), z.reshape(-1),
             exp(cx), exp(cy), exp(cz),
             idx.reshape(-1))


def kernel(pos):
    B, N, G, K = _B, _N, _G, _K
    pt = jnp.transpose(pos, (2, 0, 1))
    x, y, z = pt[0], pt[1], pt[2]
    cx, cy, cz = _fps(x, y, z)
    idx = _knn(cx, cy, cz, x, y, z)
    nbx, nby, nbz = _sc_group(x, y, z, cx, cy, cz, idx)
    neighborhood = jnp.stack(
        [nbx.reshape(B, G, K), nby.reshape(B, G, K), nbz.reshape(B, G, K)],
        axis=-1)
    centers = jnp.stack([cx, cy, cz], axis=-1)
    return neighborhood, centers
